# baseline (speedup 1.0000x reference)
import functools

import jax
import jax.numpy as jnp
from jax import lax
from jax.experimental import pallas as pl
from jax.experimental.pallas import tpu as pltpu
from jax.experimental.pallas import tpu_sc as plsc

B = 4096
CTX = 20
E = 64

NC = 2
NS = 16
NW = NC * NS
BPW = B // NW
NCHUNK = 4
CPB = BPW // NCHUNK
ROWS_PER_CHUNK = CPB * CTX


def _sc_gather_body(table_hbm, ictx_hbm, o_hbm, in_sum_hbm, out_emb_hbm,
                    idx_v, oidx_v, rows_v, acc_v, orow_v, sem, osem):
  wid = lax.axis_index("s") * NC + lax.axis_index("c")
  base = wid * BPW

  pltpu.sync_copy(ictx_hbm.at[wid], idx_v)
  pltpu.sync_copy(o_hbm.at[pl.ds(base, BPW)], oidx_v)

  ocopy = pltpu.async_copy(table_hbm.at[oidx_v], orow_v, osem)

  for c in range(NCHUNK):
    pltpu.async_copy(table_hbm.at[idx_v.at[c]], rows_v, sem).wait()

    @pl.loop(0, CPB)
    def _sum_rows(b):
      row0 = b * CTX
      for k in range(E // 16):
        cols = pl.ds(k * 16, 16)
        acc = rows_v[row0, cols]
        for t in range(1, CTX):
          acc = acc + rows_v[row0 + t, cols]
        acc_v[c * CPB + b, cols] = acc

  pltpu.sync_copy(acc_v, in_sum_hbm.at[pl.ds(base, BPW)])
  ocopy.wait()
  pltpu.sync_copy(orow_v, out_emb_hbm.at[pl.ds(base, BPW)])


def _sc_gather(table, ictx, o):
  mesh = plsc.VectorSubcoreMesh(core_axis_name="c", subcore_axis_name="s")
  f = pl.kernel(
      _sc_gather_body,
      out_type=(
          jax.ShapeDtypeStruct((B, E), jnp.float32),
          jax.ShapeDtypeStruct((B, E), jnp.float32),
      ),
      mesh=mesh,
      compiler_params=pltpu.CompilerParams(use_tc_tiling_on_sc=False),
      scratch_types=[
          pltpu.VMEM((NCHUNK, ROWS_PER_CHUNK), jnp.int32),
          pltpu.VMEM((BPW,), jnp.int32),
          pltpu.VMEM((ROWS_PER_CHUNK, E), jnp.float32),
          pltpu.VMEM((BPW, E), jnp.float32),
          pltpu.VMEM((BPW, E), jnp.float32),
          pltpu.SemaphoreType.DMA,
          pltpu.SemaphoreType.DMA,
      ],
  )
  return f(table, ictx, o)


def _tc_score_body(a_ref, b_ref, o_ref):
  a = a_ref[...] * (1.0 / CTX)
  s = lax.dot_general(
      a, b_ref[...], (((1,), (1,)), ((), ())),
      preferred_element_type=jnp.float32,
      precision=lax.Precision.DEFAULT,
  )
  o_ref[...] = jnp.minimum(s, 0.0) - jnp.log1p(jnp.exp(-jnp.abs(s)))


def _tc_score(in_sum, out_emb):
  BM = 512
  grid = (B // BM,)
  return pl.pallas_call(
      _tc_score_body,
      grid=grid,
      in_specs=[
          pl.BlockSpec((BM, E), lambda m: (m, 0)),
          pl.BlockSpec((B, E), lambda m: (0, 0)),
      ],
      out_specs=pl.BlockSpec((BM, B), lambda m: (m, 0)),
      out_shape=jax.ShapeDtypeStruct((B, B), jnp.float32),
  )(in_sum, out_emb)


@jax.jit
def kernel(i, o, table):
  ictx = i.reshape(NW, NCHUNK, ROWS_PER_CHUNK)
  in_sum, out_emb = _sc_gather(table, ictx, o)
  return _tc_score(in_sum, out_emb)
